# baseline (speedup 1.0000x reference)
import functools

import jax
import jax.numpy as jnp
from jax import lax
from jax.experimental import pallas as pl
from jax.experimental.pallas import tpu as pltpu
from jax.experimental.pallas import tpu_sc as plsc

B = 16384
D = 100
C = 128
DP = 128
LANES = 16

_info = plsc.get_sparse_core_info()
NC = _info.num_cores
NS = _info.num_subcores
NW = NC * NS
PER_W = B // NW
N_CH = PER_W // C

S_OFFS = (0, 16, 32, 48, 64, 80, 84)
W_OFFS = (0, 16, 32, 48, 64, 80, 96)


def _pack_weights(W_out, b_out):
    w3 = W_out.reshape(3, 100).astype(jnp.float32)
    wp = jnp.zeros((3, 112), jnp.float32)
    wp = wp.at[:, :96].set(w3[:, :96])
    wp = wp.at[:, 108:112].set(w3[:, 96:100])
    bias = jnp.broadcast_to(b_out.astype(jnp.float32).reshape(1), (LANES,))
    return jnp.concatenate([wp.reshape(-1), bias])


_mesh = plsc.VectorSubcoreMesh(core_axis_name="c", subcore_axis_name="s")


@functools.partial(
    pl.kernel,
    out_type=jax.ShapeDtypeStruct((B,), jnp.float32),
    mesh=_mesh,
    compiler_params=pltpu.CompilerParams(
        needs_layout_passes=False, use_tc_tiling_on_sc=False),
    scratch_types=[
        pltpu.VMEM((C,), jnp.int32),
        pltpu.VMEM((C,), jnp.int32),
        pltpu.VMEM((C, D), jnp.float32),
        pltpu.VMEM((C, D), jnp.float32),
        pltpu.VMEM((C, D), jnp.float32),
        pltpu.VMEM((C, D), jnp.float32),
        pltpu.VMEM((352,), jnp.float32),
        pltpu.VMEM((C,), jnp.float32),
        pltpu.SemaphoreType.DMA,
        pltpu.SemaphoreType.DMA,
        pltpu.SemaphoreType.DMA,
        pltpu.SemaphoreType.DMA,
    ],
)
def _sc_kernel(user_hbm, sku_hbm, t_sku, t_u1, t_u2, t_u3, wb_hbm,
               out_hbm,
               idxu, idxs, rs, r1, r2, r3, wv, ov, sm0, sm1, sm2, sm3):
    wid = lax.axis_index("s") * NC + lax.axis_index("c")
    pltpu.sync_copy(wb_hbm, wv)
    bias = wv[pl.ds(336, LANES)]
    wregs = [[wv[pl.ds(112 * t + W_OFFS[ci], LANES)] for ci in range(7)]
             for t in range(3)]
    lane = lax.iota(jnp.int32, LANES)
    last_lane = lane == (LANES - 1)

    for ch in range(N_CH):
        base = wid * PER_W + ch * C
        pltpu.sync_copy(user_hbm.at[pl.ds(base, C)], idxu)
        pltpu.sync_copy(sku_hbm.at[pl.ds(base, C)], idxs)
        cps = (pltpu.async_copy(t_sku.at[idxs], rs, sm0),
               pltpu.async_copy(t_u1.at[idxu], r1, sm1),
               pltpu.async_copy(t_u2.at[idxu], r2, sm2),
               pltpu.async_copy(t_u3.at[idxu], r3, sm3))
        for cp in cps:
            cp.wait()

        def body(g, carry):
            totals = jnp.zeros((LANES,), jnp.float32)
            for e in range(LANES):
                b = g * LANES + e
                acc = jnp.zeros((LANES,), jnp.float32)
                for ci in range(7):
                    off = S_OFFS[ci]
                    s = rs[b, pl.ds(off, LANES)]
                    for t, r in enumerate((r1, r2, r3)):
                        u = r[b, pl.ds(off, LANES)]
                        acc = acc + jnp.abs(s - u) * wregs[t][ci]
                totals = jnp.where(lane == e, jnp.sum(acc), totals)
            y = 1.0 / (1.0 + jnp.exp(-(totals + bias)))
            ov[pl.ds(g * LANES, LANES)] = y
            return carry

        lax.fori_loop(0, C // LANES, body, 0)
        pltpu.sync_copy(ov, out_hbm.at[pl.ds(base, C)])


def kernel(user, sku, sku_table, user_table1, user_table2, user_table3, W_out, b_out):
    wb = _pack_weights(W_out, b_out)
    out = _sc_kernel(user.astype(jnp.int32), sku.astype(jnp.int32),
                     sku_table, user_table1, user_table2, user_table3, wb)
    return out.reshape(B, 1)


# speedup vs baseline: 5.3235x; 5.3235x over previous
import functools

import jax
import jax.numpy as jnp
from jax import lax
from jax.experimental import pallas as pl
from jax.experimental.pallas import tpu as pltpu
from jax.experimental.pallas import tpu_sc as plsc

B = 16384
D = 100
C = 128
LANES = 16

_info = plsc.get_sparse_core_info()
NC = _info.num_cores
NS = _info.num_subcores
NW = NC * NS
PER_W = B // NW
N_CH = PER_W // C

S_OFFS = (0, 16, 32, 48, 64, 80, 84)
W_OFFS = (0, 16, 32, 48, 64, 80, 96)


def _pack_weights(W_out, b_out):
    w3 = W_out.reshape(3, 100).astype(jnp.float32)
    wp = jnp.zeros((3, 112), jnp.float32)
    wp = wp.at[:, :96].set(w3[:, :96])
    wp = wp.at[:, 108:112].set(w3[:, 96:100])
    bias = jnp.broadcast_to(b_out.astype(jnp.float32).reshape(1), (LANES,))
    return jnp.concatenate([wp.reshape(-1), bias])


_mesh = plsc.VectorSubcoreMesh(core_axis_name="c", subcore_axis_name="s")


@functools.partial(
    pl.kernel,
    out_type=jax.ShapeDtypeStruct((B,), jnp.float32),
    mesh=_mesh,
    compiler_params=pltpu.CompilerParams(needs_layout_passes=False),
    scratch_types=[
        pltpu.VMEM((C,), jnp.int32),
        pltpu.VMEM((C,), jnp.int32),
        pltpu.VMEM((C, D), jnp.float32),
        pltpu.VMEM((C, D), jnp.float32),
        pltpu.VMEM((C, D), jnp.float32),
        pltpu.VMEM((C, D), jnp.float32),
        pltpu.VMEM((352,), jnp.float32),
        pltpu.VMEM((C,), jnp.float32),
        pltpu.SemaphoreType.DMA,
        pltpu.SemaphoreType.DMA,
        pltpu.SemaphoreType.DMA,
        pltpu.SemaphoreType.DMA,
    ],
)
def _sc_kernel(user_hbm, sku_hbm, t_sku, t_u1, t_u2, t_u3, wb_hbm,
               out_hbm,
               idxu, idxs, rs, r1, r2, r3, wv, ov, sm0, sm1, sm2, sm3):
    wid = lax.axis_index("s") * NC + lax.axis_index("c")
    pltpu.sync_copy(wb_hbm, wv)
    bias = wv[pl.ds(336, LANES)]
    wregs = [[wv[pl.ds(112 * t + W_OFFS[ci], LANES)] for ci in range(7)]
             for t in range(3)]
    lane = lax.iota(jnp.int32, LANES)

    def chunk_body(ch, carry):
        base = wid * PER_W + ch * C
        pltpu.sync_copy(user_hbm.at[pl.ds(base, C)], idxu)
        pltpu.sync_copy(sku_hbm.at[pl.ds(base, C)], idxs)

        def issue_group(g, carry2):
            gb = g * LANES
            su = idxu[pl.ds(gb, LANES)]
            ss = idxs[pl.ds(gb, LANES)]
            for e in range(LANES):
                b = gb + e
                rsku = ss[e]
                rusr = su[e]
                pltpu.async_copy(t_sku.at[pl.ds(rsku, 1)],
                                 rs.at[pl.ds(b, 1)], sm0)
                pltpu.async_copy(t_u1.at[pl.ds(rusr, 1)],
                                 r1.at[pl.ds(b, 1)], sm1)
                pltpu.async_copy(t_u2.at[pl.ds(rusr, 1)],
                                 r2.at[pl.ds(b, 1)], sm2)
                pltpu.async_copy(t_u3.at[pl.ds(rusr, 1)],
                                 r3.at[pl.ds(b, 1)], sm3)
            return carry2

        lax.fori_loop(0, C // LANES, issue_group, 0)

        pltpu.make_async_copy(t_sku.at[pl.ds(0, C)], rs, sm0).wait()
        pltpu.make_async_copy(t_u1.at[pl.ds(0, C)], r1, sm1).wait()
        pltpu.make_async_copy(t_u2.at[pl.ds(0, C)], r2, sm2).wait()
        pltpu.make_async_copy(t_u3.at[pl.ds(0, C)], r3, sm3).wait()

        def body(g, carry2):
            totals = jnp.zeros((LANES,), jnp.float32)
            for e in range(LANES):
                b = g * LANES + e
                acc = jnp.zeros((LANES,), jnp.float32)
                for ci in range(7):
                    off = S_OFFS[ci]
                    s = rs[b, pl.ds(off, LANES)]
                    for t, r in enumerate((r1, r2, r3)):
                        u = r[b, pl.ds(off, LANES)]
                        acc = acc + jnp.abs(s - u) * wregs[t][ci]
                totals = jnp.where(lane == e, jnp.sum(acc), totals)
            y = 1.0 / (1.0 + jnp.exp(-(totals + bias)))
            ov[pl.ds(g * LANES, LANES)] = y
            return carry2

        lax.fori_loop(0, C // LANES, body, 0)
        pltpu.sync_copy(ov, out_hbm.at[pl.ds(base, C)])
        return carry

    lax.fori_loop(0, N_CH, chunk_body, 0)


def kernel(user, sku, sku_table, user_table1, user_table2, user_table3, W_out, b_out):
    wb = _pack_weights(W_out, b_out)
    out = _sc_kernel(user.astype(jnp.int32), sku.astype(jnp.int32),
                     sku_table, user_table1, user_table2, user_table3, wb)
    return out.reshape(B, 1)
